# baseline (speedup 1.0000x reference)
import functools

import numpy as np
import jax
import jax.numpy as jnp
from jax import lax
from jax.experimental import pallas as pl
from jax.experimental.pallas import tpu as pltpu
from jax.experimental.pallas import tpu_sc as plsc

LATENT = 64
DRAWS = 100
BATCH = 16384
BLK = 256

_KS0 = np.uint32(0)
_KS1 = np.uint32(42)
_KS2 = np.uint32(0x1BD11BDA ^ 42)
_TINY = np.float32(np.finfo(np.float32).tiny)
_ROTS = ((13, 15, 26, 6), (17, 29, 16, 24))


def _rotl(x, r):
    return jnp.left_shift(x, np.uint32(r)) | jnp.right_shift(x, np.uint32(32 - r))


def _threefry_bits(n):
    ks = (_KS0, _KS1, _KS2)
    x0 = jnp.zeros_like(n)
    x1 = n + _KS1
    for i in range(5):
        for r in _ROTS[i % 2]:
            x0 = x0 + x1
            x1 = _rotl(x1, r)
            x1 = x1 ^ x0
        x0 = x0 + ks[(i + 1) % 3]
        x1 = x1 + ks[(i + 2) % 3] + np.uint32(i + 1)
    return x0 ^ x1


def _uniform(bits):
    fb = jnp.right_shift(bits, np.uint32(9)) | np.uint32(0x3F800000)
    f = lax.bitcast_convert_type(fb, jnp.float32) - np.float32(1.0)
    return jnp.maximum(f, _TINY)


def _mxu_t(x, n):
    eye = (
        lax.broadcasted_iota(jnp.int32, (n, n), 0)
        == lax.broadcasted_iota(jnp.int32, (n, n), 1)
    ).astype(jnp.float32)
    return lax.dot_general(
        x, eye, (((0,), (0,)), ((), ())), preferred_element_type=jnp.float32
    )


def _sample_body(batch, blk, embs_ref, par_ref, out_ref):
    a = embs_ref[...]
    par = par_ref[...] != 0
    sel = jnp.where(par, a[:, LATENT:], a[:, :LATENT])
    m = jnp.max(sel, axis=1, keepdims=True)
    t = jnp.exp(sel - m)
    s = jnp.sum(t, axis=1, keepdims=True)
    probs = t / s
    nrinv = _mxu_t(np.float32(-1.0) / t, blk)

    col = lax.broadcasted_iota(jnp.int32, (LATENT, blk), 1) + pl.program_id(0) * blk
    row = lax.broadcasted_iota(jnp.int32, (LATENT, blk), 0)
    base = (col * LATENT + row).astype(jnp.uint32)
    jio = row
    stride = np.int32(batch * LATENT)

    def one_draw(n, counts):
        v = jnp.log(_uniform(_threefry_bits(n))) * nrinv
        key = (lax.bitcast_convert_type(v, jnp.int32) & np.int32(~63)) | jio
        kmin = jnp.min(key, axis=0, keepdims=True)
        return counts + (key == kmin).astype(jnp.float32)

    def body(k, counts):
        n = base + ((2 * k) * stride).astype(jnp.uint32)
        counts = one_draw(n, counts)
        return one_draw(n + stride.astype(jnp.uint32), counts)

    counts = lax.fori_loop(0, DRAWS // 2, body, jnp.zeros((LATENT, blk), jnp.float32))
    out_ref[...] = _mxu_t(counts, LATENT) * probs * np.float32(1.0 / DRAWS)


def _make_sampler(batch, blk, interpret=False):
    return pl.pallas_call(
        functools.partial(_sample_body, batch, blk),
        grid=(batch // blk,),
        in_specs=[
            pl.BlockSpec((blk, 2 * LATENT), lambda i: (i, 0)),
            pl.BlockSpec((blk, 1), lambda i: (i, 0)),
        ],
        out_specs=pl.BlockSpec((blk, LATENT), lambda i: (i, 0)),
        out_shape=jax.ShapeDtypeStruct((batch, LATENT), jnp.float32),
        interpret=interpret,
    )


_NW = 32
_BPW = BATCH // _NW


def _gather_body(table_hbm, idx_hbm, out_hbm, idx_v, rows_v, sem):
    wid = lax.axis_index("s") * 2 + lax.axis_index("c")
    base = wid * _BPW
    pltpu.sync_copy(idx_hbm.at[pl.ds(base, _BPW)], idx_v)
    pltpu.async_copy(table_hbm.at[idx_v], rows_v, sem).wait()
    pltpu.sync_copy(rows_v, out_hbm.at[pl.ds(base, _BPW)])


def _sc_gather(wide_table, idx):
    mesh = plsc.VectorSubcoreMesh(core_axis_name="c", subcore_axis_name="s")
    return pl.kernel(
        _gather_body,
        mesh=mesh,
        out_type=jax.ShapeDtypeStruct((BATCH, 2 * LATENT), jnp.float32),
        scratch_types=[
            pltpu.VMEM((_BPW,), jnp.int32),
            pltpu.VMEM((_BPW, 2 * LATENT), jnp.float32),
            pltpu.SemaphoreType.DMA,
        ],
    )(wide_table, idx)


def kernel(labels, table):
    idx = labels.astype(jnp.int32)
    wide = table.reshape(table.shape[0] // 2, 2 * LATENT)
    embs2 = _sc_gather(wide, idx >> 1)
    par = (idx & 1).reshape(BATCH, 1)
    return _make_sampler(BATCH, BLK)(embs2, par)


# speedup vs baseline: 1.0009x; 1.0009x over previous
import functools

import numpy as np
import jax
import jax.numpy as jnp
from jax import lax
from jax.experimental import pallas as pl
from jax.experimental.pallas import tpu as pltpu
from jax.experimental.pallas import tpu_sc as plsc

LATENT = 64
DRAWS = 100
BATCH = 16384
BLK = 256

_KS0 = np.uint32(0)
_KS1 = np.uint32(42)
_KS2 = np.uint32(0x1BD11BDA ^ 42)
_TINY = np.float32(np.finfo(np.float32).tiny)
_ROTS = ((13, 15, 26, 6), (17, 29, 16, 24))


def _rotl(x, r):
    return jnp.left_shift(x, np.uint32(r)) | jnp.right_shift(x, np.uint32(32 - r))


def _threefry_bits(n):
    ks = (_KS0, _KS1, _KS2)
    x0 = n + _KS1
    x1 = _rotl(x0, _ROTS[0][0]) ^ x0
    first = True
    for i in range(5):
        for r in _ROTS[i % 2]:
            if first:
                first = False
                continue
            x0 = x0 + x1
            x1 = _rotl(x1, r)
            x1 = x1 ^ x0
        x0 = x0 + ks[(i + 1) % 3]
        x1 = x1 + ks[(i + 2) % 3] + np.uint32(i + 1)
    return x0 ^ x1


def _uniform(bits):
    fb = jnp.right_shift(bits, np.uint32(9)) | np.uint32(0x3F800000)
    f = lax.bitcast_convert_type(fb, jnp.float32) - np.float32(1.0)
    return jnp.maximum(f, _TINY)


def _sample_body(batch, blk, embs_ref, par_ref, out_ref):
    a = embs_ref[...]
    par = par_ref[...] != 0
    sel = jnp.where(par, a[:, LATENT:], a[:, :LATENT])
    m = jnp.max(sel, axis=1, keepdims=True)
    t = jnp.exp(sel - m)
    s = jnp.sum(t, axis=1, keepdims=True)
    probs = t / s
    nrinv = (np.float32(-1.0) / t).T

    col = lax.broadcasted_iota(jnp.int32, (LATENT, blk), 1) + pl.program_id(0) * blk
    row = lax.broadcasted_iota(jnp.int32, (LATENT, blk), 0)
    base = (col * LATENT + row).astype(jnp.uint32)
    jio = row
    stride = np.int32(batch * LATENT)

    def one_draw(n, counts):
        v = jnp.log(_uniform(_threefry_bits(n))) * nrinv
        key = (lax.bitcast_convert_type(v, jnp.int32) & np.int32(~63)) | jio
        kmin = jnp.min(key, axis=0, keepdims=True)
        return counts + (key == kmin).astype(jnp.float32)

    def body(k, counts):
        n = base + ((2 * k) * stride).astype(jnp.uint32)
        counts = one_draw(n, counts)
        return one_draw(n + stride.astype(jnp.uint32), counts)

    counts = lax.fori_loop(0, DRAWS // 2, body, jnp.zeros((LATENT, blk), jnp.float32))
    out_ref[...] = counts.T * probs * np.float32(1.0 / DRAWS)


def _make_sampler(batch, blk, interpret=False):
    return pl.pallas_call(
        functools.partial(_sample_body, batch, blk),
        grid=(batch // blk,),
        in_specs=[
            pl.BlockSpec((blk, 2 * LATENT), lambda i: (i, 0)),
            pl.BlockSpec((blk, 1), lambda i: (i, 0)),
        ],
        out_specs=pl.BlockSpec((blk, LATENT), lambda i: (i, 0)),
        out_shape=jax.ShapeDtypeStruct((batch, LATENT), jnp.float32),
        interpret=interpret,
    )


_NW = 32
_BPW = BATCH // _NW


def _gather_body(table_hbm, idx_hbm, out_hbm, idx_v, rows_v, sem):
    wid = lax.axis_index("s") * 2 + lax.axis_index("c")
    base = wid * _BPW
    pltpu.sync_copy(idx_hbm.at[pl.ds(base, _BPW)], idx_v)
    pltpu.async_copy(table_hbm.at[idx_v], rows_v, sem).wait()
    pltpu.sync_copy(rows_v, out_hbm.at[pl.ds(base, _BPW)])


def _sc_gather(wide_table, idx):
    mesh = plsc.VectorSubcoreMesh(core_axis_name="c", subcore_axis_name="s")
    return pl.kernel(
        _gather_body,
        mesh=mesh,
        out_type=jax.ShapeDtypeStruct((BATCH, 2 * LATENT), jnp.float32),
        scratch_types=[
            pltpu.VMEM((_BPW,), jnp.int32),
            pltpu.VMEM((_BPW, 2 * LATENT), jnp.float32),
            pltpu.SemaphoreType.DMA,
        ],
    )(wide_table, idx)


def kernel(labels, table):
    idx = labels.astype(jnp.int32)
    wide = table.reshape(table.shape[0] // 2, 2 * LATENT)
    embs2 = _sc_gather(wide, idx >> 1)
    par = (idx & 1).reshape(BATCH, 1)
    return _make_sampler(BATCH, BLK)(embs2, par)


# speedup vs baseline: 1.0017x; 1.0009x over previous
import functools

import numpy as np
import jax
import jax.numpy as jnp
from jax import lax
from jax.experimental import pallas as pl
from jax.experimental.pallas import tpu as pltpu
from jax.experimental.pallas import tpu_sc as plsc

LATENT = 64
DRAWS = 100
BATCH = 16384
BLK = 256

_KS0 = np.uint32(0)
_KS1 = np.uint32(42)
_KS2 = np.uint32(0x1BD11BDA ^ 42)
_TINY = np.float32(np.finfo(np.float32).tiny)
_ROTS = ((13, 15, 26, 6), (17, 29, 16, 24))


def _rotl(x, r):
    return jnp.left_shift(x, np.uint32(r)) | jnp.right_shift(x, np.uint32(32 - r))


def _threefry_bits(n):
    ks = (_KS0, _KS1, _KS2)
    x0 = n + _KS1
    x1 = _rotl(x0, _ROTS[0][0]) ^ x0
    first = True
    for i in range(5):
        for r in _ROTS[i % 2]:
            if first:
                first = False
                continue
            x0 = x0 + x1
            x1 = _rotl(x1, r)
            x1 = x1 ^ x0
        x0 = x0 + ks[(i + 1) % 3]
        x1 = x1 + ks[(i + 2) % 3] + np.uint32(i + 1)
    return x0 ^ x1


def _uniform(bits):
    fb = jnp.right_shift(bits, np.uint32(9)) | np.uint32(0x3F800000)
    f = lax.bitcast_convert_type(fb, jnp.float32) - np.float32(1.0)
    return jnp.maximum(f, _TINY)


def _sample_body(batch, blk, embs_ref, par_ref, out_ref):
    a = embs_ref[...]
    par = par_ref[...] != 0
    sel = jnp.where(par, a[:, LATENT:], a[:, :LATENT])
    m = jnp.max(sel, axis=1, keepdims=True)
    t = jnp.exp(sel - m)
    s = jnp.sum(t, axis=1, keepdims=True)
    probs = t / s
    nrinv = (np.float32(-1.0) / t).T

    col = lax.broadcasted_iota(jnp.int32, (LATENT, blk), 1) + pl.program_id(0) * blk
    row = lax.broadcasted_iota(jnp.int32, (LATENT, blk), 0)
    base = (col * LATENT + row).astype(jnp.uint32)
    jio = row
    stride = np.int32(batch * LATENT)

    def one_draw(n, counts):
        v = jnp.log(_uniform(_threefry_bits(n))) * nrinv
        key = (lax.bitcast_convert_type(v, jnp.int32) & np.int32(~63)) | jio
        kmin = jnp.min(key, axis=0, keepdims=True)
        return counts + (key == kmin).astype(jnp.float32)

    def body(k, counts):
        n = base + ((2 * k) * stride).astype(jnp.uint32)
        counts = one_draw(n, counts)
        return one_draw(n + stride.astype(jnp.uint32), counts)

    counts = lax.fori_loop(0, DRAWS // 2, body, jnp.zeros((LATENT, blk), jnp.float32))
    out_ref[...] = counts.T * probs * np.float32(1.0 / DRAWS)


def _make_sampler(batch, blk, interpret=False):
    return pl.pallas_call(
        functools.partial(_sample_body, batch, blk),
        grid=(batch // blk,),
        in_specs=[
            pl.BlockSpec((blk, 2 * LATENT), lambda i: (i, 0)),
            pl.BlockSpec((blk, 1), lambda i: (i, 0)),
        ],
        out_specs=pl.BlockSpec((blk, LATENT), lambda i: (i, 0)),
        out_shape=jax.ShapeDtypeStruct((batch, LATENT), jnp.float32),
        compiler_params=pltpu.CompilerParams(
            dimension_semantics=("parallel",)
        ),
        interpret=interpret,
    )


_NW = 32
_BPW = BATCH // _NW


def _gather_body(table_hbm, idx_hbm, out_hbm, idx_v, rows_v, sem):
    wid = lax.axis_index("s") * 2 + lax.axis_index("c")
    base = wid * _BPW
    pltpu.sync_copy(idx_hbm.at[pl.ds(base, _BPW)], idx_v)
    pltpu.async_copy(table_hbm.at[idx_v], rows_v, sem).wait()
    pltpu.sync_copy(rows_v, out_hbm.at[pl.ds(base, _BPW)])


def _sc_gather(wide_table, idx):
    mesh = plsc.VectorSubcoreMesh(core_axis_name="c", subcore_axis_name="s")
    return pl.kernel(
        _gather_body,
        mesh=mesh,
        out_type=jax.ShapeDtypeStruct((BATCH, 2 * LATENT), jnp.float32),
        scratch_types=[
            pltpu.VMEM((_BPW,), jnp.int32),
            pltpu.VMEM((_BPW, 2 * LATENT), jnp.float32),
            pltpu.SemaphoreType.DMA,
        ],
    )(wide_table, idx)


def kernel(labels, table):
    idx = labels.astype(jnp.int32)
    wide = table.reshape(table.shape[0] // 2, 2 * LATENT)
    embs2 = _sc_gather(wide, idx >> 1)
    par = (idx & 1).reshape(BATCH, 1)
    return _make_sampler(BATCH, BLK)(embs2, par)


# speedup vs baseline: 1.0186x; 1.0169x over previous
import functools

import numpy as np
import jax
import jax.numpy as jnp
from jax import lax
from jax.experimental import pallas as pl
from jax.experimental.pallas import tpu as pltpu
from jax.experimental.pallas import tpu_sc as plsc

LATENT = 64
DRAWS = 100
BATCH = 16384
BLK = 256

_KS0 = np.uint32(0)
_KS1 = np.uint32(42)
_KS2 = np.uint32(0x1BD11BDA ^ 42)
_TINY = np.float32(np.finfo(np.float32).tiny)
_ROTS = ((13, 15, 26, 6), (17, 29, 16, 24))


def _rotl(x, r):
    return jnp.left_shift(x, np.uint32(r)) | jnp.right_shift(x, np.uint32(32 - r))


def _threefry_bits(n):
    ks = (_KS0, _KS1, _KS2)
    x0 = n + _KS1
    x1 = _rotl(x0, _ROTS[0][0]) ^ x0
    first = True
    for i in range(5):
        for r in _ROTS[i % 2]:
            if first:
                first = False
                continue
            x0 = x0 + x1
            x1 = _rotl(x1, r)
            x1 = x1 ^ x0
        x0 = x0 + ks[(i + 1) % 3]
        x1 = x1 + ks[(i + 2) % 3] + np.uint32(i + 1)
    return x0 ^ x1


def _uniform(bits):
    fb = jnp.right_shift(bits, np.uint32(9)) | np.uint32(0x3F800000)
    f = lax.bitcast_convert_type(fb, jnp.float32) - np.float32(1.0)
    return jnp.maximum(f, _TINY)


def _sample_body(batch, blk, embs_ref, par_ref, out_ref):
    a = embs_ref[...]
    par = par_ref[...] != 0
    lt = jnp.where(par, a[LATENT:, :], a[:LATENT, :])
    m = jnp.max(lt, axis=0, keepdims=True)
    t = jnp.exp(lt - m)
    s = jnp.sum(t, axis=0, keepdims=True)
    probs = t / s
    nrinv = np.float32(-1.0) / t

    col = lax.broadcasted_iota(jnp.int32, (LATENT, blk), 1) + pl.program_id(0) * blk
    row = lax.broadcasted_iota(jnp.int32, (LATENT, blk), 0)
    base = (col * LATENT + row).astype(jnp.uint32)
    jio = row
    stride = np.int32(batch * LATENT)

    def one_draw(n, counts):
        v = jnp.log(_uniform(_threefry_bits(n))) * nrinv
        key = (lax.bitcast_convert_type(v, jnp.int32) & np.int32(~63)) | jio
        kmin = jnp.min(key, axis=0, keepdims=True)
        return counts + (key == kmin).astype(jnp.float32)

    def body(k, counts):
        n = base + ((2 * k) * stride).astype(jnp.uint32)
        counts = one_draw(n, counts)
        return one_draw(n + stride.astype(jnp.uint32), counts)

    counts = lax.fori_loop(0, DRAWS // 2, body, jnp.zeros((LATENT, blk), jnp.float32))
    out_ref[...] = counts * probs * np.float32(1.0 / DRAWS)


def _make_sampler(batch, blk, interpret=False):
    return pl.pallas_call(
        functools.partial(_sample_body, batch, blk),
        grid=(batch // blk,),
        in_specs=[
            pl.BlockSpec((2 * LATENT, blk), lambda i: (0, i)),
            pl.BlockSpec((1, blk), lambda i: (0, i)),
        ],
        out_specs=pl.BlockSpec((LATENT, blk), lambda i: (0, i)),
        out_shape=jax.ShapeDtypeStruct((LATENT, batch), jnp.float32),
        compiler_params=pltpu.CompilerParams(
            dimension_semantics=("parallel",)
        ),
        interpret=interpret,
    )


_NW = 32
_BPW = BATCH // _NW


def _gather_body(table_hbm, idx_hbm, out_hbm, idx_v, rows_v, sem):
    wid = lax.axis_index("s") * 2 + lax.axis_index("c")
    base = wid * _BPW
    pltpu.sync_copy(idx_hbm.at[pl.ds(base, _BPW)], idx_v)
    pltpu.async_copy(table_hbm.at[idx_v], rows_v, sem).wait()
    pltpu.sync_copy(rows_v, out_hbm.at[pl.ds(base, _BPW)])


def _sc_gather(wide_table, idx):
    mesh = plsc.VectorSubcoreMesh(core_axis_name="c", subcore_axis_name="s")
    return pl.kernel(
        _gather_body,
        mesh=mesh,
        out_type=jax.ShapeDtypeStruct((BATCH, 2 * LATENT), jnp.float32),
        scratch_types=[
            pltpu.VMEM((_BPW,), jnp.int32),
            pltpu.VMEM((_BPW, 2 * LATENT), jnp.float32),
            pltpu.SemaphoreType.DMA,
        ],
    )(wide_table, idx)


def kernel(labels, table):
    idx = labels.astype(jnp.int32)
    wide = table.reshape(table.shape[0] // 2, 2 * LATENT)
    embs2 = _sc_gather(wide, idx >> 1)
    par = (idx & 1).reshape(1, BATCH)
    out = _make_sampler(BATCH, BLK)(embs2.T, par)
    return out.T
